# Initial kernel scaffold; baseline (speedup 1.0000x reference)
import jax
import jax.numpy as jnp
from jax.experimental import pallas as pl


def kernel(x, edge_index, W1, b1, W2, b2):
    raise NotImplementedError("write your pallas kernel here")


# speedup vs baseline: 25.5933x; 25.5933x over previous
import functools

import jax
import jax.numpy as jnp
from jax import lax
from jax.experimental import pallas as pl
from jax.experimental.pallas import tpu as pltpu
from jax.experimental.pallas import tpu_sc as plsc

N = 10000
E = 320000
D = 128
H = 16
C = 40
CP = 48

NC = 2
NS = 16
NW = NC * NS
EW = E // NW
KB = 125
BB = 80
RPT = N // NS

_mesh = plsc.VectorSubcoreMesh(core_axis_name="c", subcore_axis_name="s")


def _sc_degree(dst_blk, ones_rows, zero_rows):
    @functools.partial(
        pl.kernel,
        out_type=jax.ShapeDtypeStruct((NC, N, H), jnp.float32),
        mesh=_mesh,
        compiler_params=pltpu.CompilerParams(use_tc_tiling_on_sc=False),
        scratch_types=[
            pltpu.VMEM((KB, BB), jnp.int32),
            pltpu.VMEM((BB, H), jnp.float32),
            pltpu.VMEM((RPT, H), jnp.float32),
            pltpu.VMEM_SHARED((N, H), jnp.float32),
            pltpu.SemaphoreType.DMA,
        ],
    )
    def deg_kernel(dst_hbm, ones_hbm, zeros_hbm, out_hbm, didx, ones_v, bounce, acc, sem):
        c = lax.axis_index("c")
        s = lax.axis_index("s")
        wid = c * NS + s
        pltpu.sync_copy(zeros_hbm, bounce)
        pltpu.sync_copy(bounce, acc.at[pl.ds(s * RPT, RPT)])
        pltpu.sync_copy(ones_hbm, ones_v)
        pltpu.sync_copy(dst_hbm.at[wid], didx)
        plsc.subcore_barrier()

        def body(j, _):
            pltpu.sync_copy(ones_v, acc.at[didx.at[j]], add=True)
            return _

        lax.fori_loop(0, KB, body, None)
        plsc.subcore_barrier()
        pltpu.sync_copy(acc.at[pl.ds(s * RPT, RPT)], bounce)
        pltpu.sync_copy(bounce, out_hbm.at[c, pl.ds(s * RPT, RPT)])

    return deg_kernel(dst_blk, ones_rows, zero_rows)


def _sc_message_pass(width):
    @functools.partial(
        pl.kernel,
        out_type=jax.ShapeDtypeStruct((NC, N, width), jnp.float32),
        mesh=_mesh,
        compiler_params=pltpu.CompilerParams(use_tc_tiling_on_sc=False),
        scratch_types=[
            pltpu.VMEM((KB, BB), jnp.int32),
            pltpu.VMEM((KB, BB), jnp.int32),
            pltpu.VMEM((BB, width), jnp.float32),
            pltpu.VMEM((RPT, width), jnp.float32),
            pltpu.VMEM_SHARED((N, width), jnp.float32),
            pltpu.SemaphoreType.DMA,
        ],
    )
    def mp_kernel(g_hbm, src_hbm, dst_hbm, zeros_hbm, out_hbm,
                  sidx, didx, rows, bounce, acc, sem):
        c = lax.axis_index("c")
        s = lax.axis_index("s")
        wid = c * NS + s
        pltpu.sync_copy(zeros_hbm, bounce)
        pltpu.sync_copy(bounce, acc.at[pl.ds(s * RPT, RPT)])
        pltpu.sync_copy(src_hbm.at[wid], sidx)
        pltpu.sync_copy(dst_hbm.at[wid], didx)
        plsc.subcore_barrier()

        def body(j, _):
            pltpu.async_copy(g_hbm.at[sidx.at[j]], rows, sem).wait()
            pltpu.sync_copy(rows, acc.at[didx.at[j]], add=True)
            return _

        lax.fori_loop(0, KB, body, None)
        plsc.subcore_barrier()
        pltpu.sync_copy(acc.at[pl.ds(s * RPT, RPT)], bounce)
        pltpu.sync_copy(bounce, out_hbm.at[c, pl.ds(s * RPT, RPT)])

    return mp_kernel


_R = 1000


def _tc_matmul(x, w1):
    def body(x_ref, w_ref, o_ref):
        o_ref[...] = jnp.dot(x_ref[...], w_ref[...],
                             preferred_element_type=jnp.float32)

    return pl.pallas_call(
        body,
        grid=(N // _R,),
        in_specs=[
            pl.BlockSpec((_R, D), lambda i: (i, 0)),
            pl.BlockSpec((D, H), lambda i: (0, 0)),
        ],
        out_specs=pl.BlockSpec((_R, H), lambda i: (i, 0)),
        out_shape=jax.ShapeDtypeStruct((N, H), jnp.float32),
    )(x, w1)


def _tc_norm(dacc, h1):
    def body(d_ref, h_ref, g_ref, dis_ref):
        deg = d_ref[0] + d_ref[1] + 1.0
        dis = lax.rsqrt(deg)
        dis_ref[...] = dis
        g_ref[...] = dis * h_ref[...]

    return pl.pallas_call(
        body,
        grid=(N // _R,),
        in_specs=[
            pl.BlockSpec((NC, _R, H), lambda i: (0, i, 0)),
            pl.BlockSpec((_R, H), lambda i: (i, 0)),
        ],
        out_specs=[
            pl.BlockSpec((_R, H), lambda i: (i, 0)),
            pl.BlockSpec((_R, H), lambda i: (i, 0)),
        ],
        out_shape=[
            jax.ShapeDtypeStruct((N, H), jnp.float32),
            jax.ShapeDtypeStruct((N, H), jnp.float32),
        ],
    )(dacc, h1)


def _tc_layer1(s1, g1, dis, b1, w2p):
    def body(s_ref, g_ref, dis_ref, b_ref, w_ref, o_ref):
        agg = dis_ref[...] * (s_ref[0] + s_ref[1] + g_ref[...]) + b_ref[...]
        z1 = jnp.maximum(agg, 0.0)
        h2 = jnp.dot(z1, w_ref[...], preferred_element_type=jnp.float32)
        o_ref[...] = dis_ref[:, :1] * h2

    return pl.pallas_call(
        body,
        grid=(N // _R,),
        in_specs=[
            pl.BlockSpec((NC, _R, H), lambda i: (0, i, 0)),
            pl.BlockSpec((_R, H), lambda i: (i, 0)),
            pl.BlockSpec((_R, H), lambda i: (i, 0)),
            pl.BlockSpec((1, H), lambda i: (0, 0)),
            pl.BlockSpec((H, CP), lambda i: (0, 0)),
        ],
        out_specs=pl.BlockSpec((_R, CP), lambda i: (i, 0)),
        out_shape=jax.ShapeDtypeStruct((N, CP), jnp.float32),
    )(s1, g1, dis, b1, w2p)


def _tc_final(s2, g2, dis, b2p):
    def body(s_ref, g_ref, dis_ref, b_ref, o_ref):
        agg = dis_ref[:, :1] * (s_ref[0] + s_ref[1] + g_ref[...]) + b_ref[...]
        logits = agg[:, :C]
        m = jnp.max(logits, axis=1, keepdims=True)
        t = logits - m
        lse = jnp.log(jnp.sum(jnp.exp(t), axis=1, keepdims=True))
        o_ref[...] = t - lse

    return pl.pallas_call(
        body,
        grid=(N // _R,),
        in_specs=[
            pl.BlockSpec((NC, _R, CP), lambda i: (0, i, 0)),
            pl.BlockSpec((_R, CP), lambda i: (i, 0)),
            pl.BlockSpec((_R, H), lambda i: (i, 0)),
            pl.BlockSpec((1, CP), lambda i: (0, 0)),
        ],
        out_specs=pl.BlockSpec((_R, C), lambda i: (i, 0)),
        out_shape=jax.ShapeDtypeStruct((N, C), jnp.float32),
    )(s2, g2, dis, b2p)


def kernel(x, edge_index, W1, b1, W2, b2):
    ei = edge_index.astype(jnp.int32)
    src = ei[0].reshape(NW, KB, BB)
    dst = ei[1].reshape(NW, KB, BB)

    ones_rows = jnp.ones((BB, H), jnp.float32)
    zeros16 = jnp.zeros((RPT, H), jnp.float32)
    zeros48 = jnp.zeros((RPT, CP), jnp.float32)
    w2p = jnp.pad(W2, ((0, 0), (0, CP - C)))
    b2p = jnp.pad(b2, (0, CP - C)).reshape(1, CP)

    h1 = _tc_matmul(x, W1)
    dacc = _sc_degree(dst, ones_rows, zeros16)
    g1, dis = _tc_norm(dacc, h1)
    s1 = _sc_message_pass(H)(g1, src, dst, zeros16)
    g2 = _tc_layer1(s1, g1, dis, b1.reshape(1, H), w2p)
    s2 = _sc_message_pass(CP)(g2, src, dst, zeros48)
    return _tc_final(s2, g2, dis, b2p)


# speedup vs baseline: 47.4066x; 1.8523x over previous
import functools

import jax
import jax.numpy as jnp
from jax import lax
from jax.experimental import pallas as pl
from jax.experimental.pallas import tpu as pltpu
from jax.experimental.pallas import tpu_sc as plsc

N = 10000
E = 320000
D = 128
H = 16
C = 40
CP = 48

NC = 2
NS = 16
NW = NC * NS
EW = E // NW
KB = 125
BB = 80
RPT = N // NS
P = 5

_mesh = plsc.VectorSubcoreMesh(core_axis_name="c", subcore_axis_name="s")


def _sc_degree(dst_blk, ones_rows, zero_rows):
    @functools.partial(
        pl.kernel,
        out_type=jax.ShapeDtypeStruct((NC, N, H), jnp.float32),
        mesh=_mesh,
        compiler_params=pltpu.CompilerParams(use_tc_tiling_on_sc=False),
        scratch_types=[
            pltpu.VMEM((KB, BB), jnp.int32),
            pltpu.VMEM((BB, H), jnp.float32),
            pltpu.VMEM((RPT, H), jnp.float32),
            pltpu.VMEM_SHARED((N, H), jnp.float32),
            pltpu.SemaphoreType.DMA((P,)),
        ],
    )
    def deg_kernel(dst_hbm, ones_hbm, zeros_hbm, out_hbm, didx, ones_v, bounce, acc, sem):
        c = lax.axis_index("c")
        s = lax.axis_index("s")
        wid = c * NS + s
        pltpu.sync_copy(zeros_hbm, bounce)
        pltpu.sync_copy(bounce, acc.at[pl.ds(s * RPT, RPT)])
        pltpu.sync_copy(ones_hbm, ones_v)
        pltpu.sync_copy(dst_hbm.at[wid], didx)
        plsc.subcore_barrier()

        def body(jj, _):
            for b in range(P):
                j = jj * P + b
                pltpu.async_copy(ones_v, acc.at[didx.at[j]], sem.at[b], add=True)

                @pl.when(jj >= 1)
                def _wait():
                    pltpu.make_async_copy(
                        ones_v, acc.at[didx.at[j - P]], sem.at[b]).wait()

            return _

        lax.fori_loop(0, KB // P, body, None)
        for b in range(P):
            pltpu.make_async_copy(
                ones_v, acc.at[didx.at[KB - P + b]], sem.at[b]).wait()
        plsc.subcore_barrier()
        pltpu.sync_copy(acc.at[pl.ds(s * RPT, RPT)], bounce)
        pltpu.sync_copy(bounce, out_hbm.at[c, pl.ds(s * RPT, RPT)])

    return deg_kernel(dst_blk, ones_rows, zero_rows)


def _sc_message_pass(width):
    @functools.partial(
        pl.kernel,
        out_type=jax.ShapeDtypeStruct((NC, N, width), jnp.float32),
        mesh=_mesh,
        compiler_params=pltpu.CompilerParams(use_tc_tiling_on_sc=False),
        scratch_types=[
            pltpu.VMEM((KB, BB), jnp.int32),
            pltpu.VMEM((KB, BB), jnp.int32),
            pltpu.VMEM((P, BB, width), jnp.float32),
            pltpu.VMEM((RPT, width), jnp.float32),
            pltpu.VMEM_SHARED((N, width), jnp.float32),
            pltpu.SemaphoreType.DMA((P,)),
        ],
    )
    def mp_kernel(g_hbm, src_hbm, dst_hbm, zeros_hbm, out_hbm,
                  sidx, didx, rows, bounce, acc, sem):
        c = lax.axis_index("c")
        s = lax.axis_index("s")
        wid = c * NS + s
        pltpu.sync_copy(zeros_hbm, bounce)
        pltpu.sync_copy(bounce, acc.at[pl.ds(s * RPT, RPT)])
        pltpu.sync_copy(src_hbm.at[wid], sidx)
        pltpu.sync_copy(dst_hbm.at[wid], didx)
        plsc.subcore_barrier()

        for b in range(P):
            pltpu.async_copy(g_hbm.at[sidx.at[b]], rows.at[b], sem.at[b])

        def body(jj, _):
            for b in range(P):
                j = jj * P + b
                pltpu.make_async_copy(
                    g_hbm.at[sidx.at[j]], rows.at[b], sem.at[b]).wait()
                pltpu.sync_copy(rows.at[b], acc.at[didx.at[j]], add=True)

                @pl.when(j + P < KB)
                def _next():
                    pltpu.async_copy(
                        g_hbm.at[sidx.at[j + P]], rows.at[b], sem.at[b])

            return _

        lax.fori_loop(0, KB // P, body, None)
        plsc.subcore_barrier()
        pltpu.sync_copy(acc.at[pl.ds(s * RPT, RPT)], bounce)
        pltpu.sync_copy(bounce, out_hbm.at[c, pl.ds(s * RPT, RPT)])

    return mp_kernel


_R = 1000


def _tc_matmul(x, w1):
    def body(x_ref, w_ref, o_ref):
        o_ref[...] = jnp.dot(x_ref[...], w_ref[...],
                             preferred_element_type=jnp.float32)

    return pl.pallas_call(
        body,
        grid=(N // _R,),
        in_specs=[
            pl.BlockSpec((_R, D), lambda i: (i, 0)),
            pl.BlockSpec((D, H), lambda i: (0, 0)),
        ],
        out_specs=pl.BlockSpec((_R, H), lambda i: (i, 0)),
        out_shape=jax.ShapeDtypeStruct((N, H), jnp.float32),
    )(x, w1)


def _tc_norm(dacc, h1):
    def body(d_ref, h_ref, g_ref, dis_ref):
        deg = d_ref[0] + d_ref[1] + 1.0
        dis = lax.rsqrt(deg)
        dis_ref[...] = dis
        g_ref[...] = dis * h_ref[...]

    return pl.pallas_call(
        body,
        grid=(N // _R,),
        in_specs=[
            pl.BlockSpec((NC, _R, H), lambda i: (0, i, 0)),
            pl.BlockSpec((_R, H), lambda i: (i, 0)),
        ],
        out_specs=[
            pl.BlockSpec((_R, H), lambda i: (i, 0)),
            pl.BlockSpec((_R, H), lambda i: (i, 0)),
        ],
        out_shape=[
            jax.ShapeDtypeStruct((N, H), jnp.float32),
            jax.ShapeDtypeStruct((N, H), jnp.float32),
        ],
    )(dacc, h1)


def _tc_layer1(s1, g1, dis, b1, w2p):
    def body(s_ref, g_ref, dis_ref, b_ref, w_ref, o_ref):
        agg = dis_ref[...] * (s_ref[0] + s_ref[1] + g_ref[...]) + b_ref[...]
        z1 = jnp.maximum(agg, 0.0)
        h2 = jnp.dot(z1, w_ref[...], preferred_element_type=jnp.float32)
        o_ref[...] = dis_ref[:, :1] * h2

    return pl.pallas_call(
        body,
        grid=(N // _R,),
        in_specs=[
            pl.BlockSpec((NC, _R, H), lambda i: (0, i, 0)),
            pl.BlockSpec((_R, H), lambda i: (i, 0)),
            pl.BlockSpec((_R, H), lambda i: (i, 0)),
            pl.BlockSpec((1, H), lambda i: (0, 0)),
            pl.BlockSpec((H, CP), lambda i: (0, 0)),
        ],
        out_specs=pl.BlockSpec((_R, CP), lambda i: (i, 0)),
        out_shape=jax.ShapeDtypeStruct((N, CP), jnp.float32),
    )(s1, g1, dis, b1, w2p)


def _tc_final(s2, g2, dis, b2p):
    def body(s_ref, g_ref, dis_ref, b_ref, o_ref):
        agg = dis_ref[:, :1] * (s_ref[0] + s_ref[1] + g_ref[...]) + b_ref[...]
        logits = agg[:, :C]
        m = jnp.max(logits, axis=1, keepdims=True)
        t = logits - m
        lse = jnp.log(jnp.sum(jnp.exp(t), axis=1, keepdims=True))
        o_ref[...] = t - lse

    return pl.pallas_call(
        body,
        grid=(N // _R,),
        in_specs=[
            pl.BlockSpec((NC, _R, CP), lambda i: (0, i, 0)),
            pl.BlockSpec((_R, CP), lambda i: (i, 0)),
            pl.BlockSpec((_R, H), lambda i: (i, 0)),
            pl.BlockSpec((1, CP), lambda i: (0, 0)),
        ],
        out_specs=pl.BlockSpec((_R, C), lambda i: (i, 0)),
        out_shape=jax.ShapeDtypeStruct((N, C), jnp.float32),
    )(s2, g2, dis, b2p)


def kernel(x, edge_index, W1, b1, W2, b2):
    ei = edge_index.astype(jnp.int32)
    src = ei[0].reshape(NW, KB, BB)
    dst = ei[1].reshape(NW, KB, BB)

    ones_rows = jnp.ones((BB, H), jnp.float32)
    zeros16 = jnp.zeros((RPT, H), jnp.float32)
    zeros48 = jnp.zeros((RPT, CP), jnp.float32)
    w2p = jnp.pad(W2, ((0, 0), (0, CP - C)))
    b2p = jnp.pad(b2, (0, CP - C)).reshape(1, CP)

    h1 = _tc_matmul(x, W1)
    dacc = _sc_degree(dst, ones_rows, zeros16)
    g1, dis = _tc_norm(dacc, h1)
    s1 = _sc_message_pass(H)(g1, src, dst, zeros16)
    g2 = _tc_layer1(s1, g1, dis, b1.reshape(1, H), w2p)
    s2 = _sc_message_pass(CP)(g2, src, dst, zeros48)
    return _tc_final(s2, g2, dis, b2p)


# speedup vs baseline: 51.7635x; 1.0919x over previous
import functools

import jax
import jax.numpy as jnp
from jax import lax
from jax.experimental import pallas as pl
from jax.experimental.pallas import tpu as pltpu
from jax.experimental.pallas import tpu_sc as plsc

N = 10000
E = 320000
D = 128
H = 16
C = 40
CP = 48

NC = 2
NS = 16
NW = NC * NS
EW = E // NW
KB = 125
BB = 80
RPT = N // NS
P = 5

_mesh = plsc.VectorSubcoreMesh(core_axis_name="c", subcore_axis_name="s")


def _sc_degree(dst_blk, ones_rows, zero_rows):
    @functools.partial(
        pl.kernel,
        out_type=jax.ShapeDtypeStruct((NC, N, H), jnp.float32),
        mesh=_mesh,
        compiler_params=pltpu.CompilerParams(use_tc_tiling_on_sc=False),
        scratch_types=[
            pltpu.VMEM((KB, BB), jnp.int32),
            pltpu.VMEM((BB, H), jnp.float32),
            pltpu.VMEM((RPT, H), jnp.float32),
            pltpu.VMEM_SHARED((N, H), jnp.float32),
            pltpu.SemaphoreType.DMA((P,)),
        ],
    )
    def deg_kernel(dst_hbm, ones_hbm, zeros_hbm, out_hbm, didx, ones_v, bounce, acc, sem):
        c = lax.axis_index("c")
        s = lax.axis_index("s")
        wid = c * NS + s
        pltpu.sync_copy(zeros_hbm, bounce)
        pltpu.sync_copy(bounce, acc.at[pl.ds(s * RPT, RPT)])
        pltpu.sync_copy(ones_hbm, ones_v)
        pltpu.sync_copy(dst_hbm.at[wid], didx)
        plsc.subcore_barrier()

        def body(jj, _):
            for b in range(P):
                j = jj * P + b
                pltpu.async_copy(ones_v, acc.at[didx.at[j]], sem.at[b], add=True)

                @pl.when(jj >= 1)
                def _wait():
                    pltpu.make_async_copy(
                        ones_v, acc.at[didx.at[j - P]], sem.at[b]).wait()

            return _

        lax.fori_loop(0, KB // P, body, None)
        for b in range(P):
            pltpu.make_async_copy(
                ones_v, acc.at[didx.at[KB - P + b]], sem.at[b]).wait()
        plsc.subcore_barrier()
        pltpu.sync_copy(acc.at[pl.ds(s * RPT, RPT)], bounce)
        pltpu.sync_copy(bounce, out_hbm.at[c, pl.ds(s * RPT, RPT)])

    return deg_kernel(dst_blk, ones_rows, zero_rows)


def _sc_message_pass(width):
    @functools.partial(
        pl.kernel,
        out_type=jax.ShapeDtypeStruct((NC, N, width), jnp.float32),
        mesh=_mesh,
        compiler_params=pltpu.CompilerParams(use_tc_tiling_on_sc=False),
        scratch_types=[
            pltpu.VMEM((KB, BB), jnp.int32),
            pltpu.VMEM((KB, BB), jnp.int32),
            pltpu.VMEM((P, BB, width), jnp.float32),
            pltpu.VMEM((RPT, width), jnp.float32),
            pltpu.VMEM_SHARED((N, width), jnp.float32),
            pltpu.SemaphoreType.DMA((P,)),
        ],
    )
    def mp_kernel(g_hbm, src_hbm, dst_hbm, zeros_hbm, out_hbm,
                  sidx, didx, rows, bounce, acc, sem):
        c = lax.axis_index("c")
        s = lax.axis_index("s")
        wid = c * NS + s
        pltpu.sync_copy(zeros_hbm, bounce)
        pltpu.sync_copy(bounce, acc.at[pl.ds(s * RPT, RPT)])
        pltpu.sync_copy(src_hbm.at[wid], sidx)
        pltpu.sync_copy(dst_hbm.at[wid], didx)
        plsc.subcore_barrier()

        for b in range(P):
            pltpu.async_copy(g_hbm.at[sidx.at[b]], rows.at[b], sem.at[b])

        def body(jj, _):
            for b in range(P):
                j = jj * P + b
                pltpu.make_async_copy(
                    g_hbm.at[sidx.at[j]], rows.at[b], sem.at[b]).wait()
                pltpu.sync_copy(rows.at[b], acc.at[didx.at[j]], add=True)

                @pl.when(j + P < KB)
                def _next():
                    pltpu.async_copy(
                        g_hbm.at[sidx.at[j + P]], rows.at[b], sem.at[b])

            return _

        lax.fori_loop(0, KB // P, body, None)
        plsc.subcore_barrier()
        pltpu.sync_copy(acc.at[pl.ds(s * RPT, RPT)], bounce)
        pltpu.sync_copy(bounce, out_hbm.at[c, pl.ds(s * RPT, RPT)])

    return mp_kernel


_R = 1000


def _tc_matmul(x, w1):
    def body(x_ref, w_ref, o_ref):
        o_ref[...] = jnp.dot(x_ref[...], w_ref[...],
                             preferred_element_type=jnp.float32)

    return pl.pallas_call(
        body,
        grid=(N // _R,),
        in_specs=[
            pl.BlockSpec((_R, D), lambda i: (i, 0)),
            pl.BlockSpec((D, H), lambda i: (0, 0)),
        ],
        out_specs=pl.BlockSpec((_R, H), lambda i: (i, 0)),
        out_shape=jax.ShapeDtypeStruct((N, H), jnp.float32),
    )(x, w1)


def _tc_norm(dacc, h1):
    def body(d_ref, h_ref, g_ref, dis_ref):
        deg = d_ref[0] + d_ref[1] + 1.0
        dis = lax.rsqrt(deg)
        dis_ref[...] = dis
        g_ref[...] = dis * h_ref[...]

    return pl.pallas_call(
        body,
        grid=(N // _R,),
        in_specs=[
            pl.BlockSpec((NC, _R, H), lambda i: (0, i, 0)),
            pl.BlockSpec((_R, H), lambda i: (i, 0)),
        ],
        out_specs=[
            pl.BlockSpec((_R, H), lambda i: (i, 0)),
            pl.BlockSpec((_R, H), lambda i: (i, 0)),
        ],
        out_shape=[
            jax.ShapeDtypeStruct((N, H), jnp.float32),
            jax.ShapeDtypeStruct((N, H), jnp.float32),
        ],
    )(dacc, h1)


def _tc_layer1(s1, g1, dis, b1):
    def body(s_ref, g_ref, dis_ref, b_ref, o_ref):
        agg = dis_ref[...] * (s_ref[0] + s_ref[1] + g_ref[...]) + b_ref[...]
        o_ref[...] = dis_ref[...] * jnp.maximum(agg, 0.0)

    return pl.pallas_call(
        body,
        grid=(N // _R,),
        in_specs=[
            pl.BlockSpec((NC, _R, H), lambda i: (0, i, 0)),
            pl.BlockSpec((_R, H), lambda i: (i, 0)),
            pl.BlockSpec((_R, H), lambda i: (i, 0)),
            pl.BlockSpec((1, H), lambda i: (0, 0)),
        ],
        out_specs=pl.BlockSpec((_R, H), lambda i: (i, 0)),
        out_shape=jax.ShapeDtypeStruct((N, H), jnp.float32),
    )(s1, g1, dis, b1)


def _tc_final(s2, g2, dis, w2, b2):
    def body(s_ref, g_ref, dis_ref, w_ref, b_ref, o_ref):
        pre = jnp.dot(s_ref[0] + s_ref[1] + g_ref[...], w_ref[...],
                      preferred_element_type=jnp.float32)
        logits = dis_ref[:, :1] * pre + b_ref[...]
        m = jnp.max(logits, axis=1, keepdims=True)
        t = logits - m
        lse = jnp.log(jnp.sum(jnp.exp(t), axis=1, keepdims=True))
        o_ref[...] = t - lse

    return pl.pallas_call(
        body,
        grid=(N // _R,),
        in_specs=[
            pl.BlockSpec((NC, _R, H), lambda i: (0, i, 0)),
            pl.BlockSpec((_R, H), lambda i: (i, 0)),
            pl.BlockSpec((_R, H), lambda i: (i, 0)),
            pl.BlockSpec((H, C), lambda i: (0, 0)),
            pl.BlockSpec((1, C), lambda i: (0, 0)),
        ],
        out_specs=pl.BlockSpec((_R, C), lambda i: (i, 0)),
        out_shape=jax.ShapeDtypeStruct((N, C), jnp.float32),
    )(s2, g2, dis, w2, b2)


def kernel(x, edge_index, W1, b1, W2, b2):
    ei = edge_index.astype(jnp.int32)
    src = ei[0].reshape(NW, KB, BB)
    dst = ei[1].reshape(NW, KB, BB)

    ones_rows = jnp.ones((BB, H), jnp.float32)
    zeros16 = jnp.zeros((RPT, H), jnp.float32)

    h1 = _tc_matmul(x, W1)
    dacc = _sc_degree(dst, ones_rows, zeros16)
    g1, dis = _tc_norm(dacc, h1)
    mp16 = _sc_message_pass(H)
    s1 = mp16(g1, src, dst, zeros16)
    g2 = _tc_layer1(s1, g1, dis, b1.reshape(1, H))
    s2 = mp16(g2, src, dst, zeros16)
    return _tc_final(s2, g2, dis, W2, b2.reshape(1, C))
